# baseline (speedup 1.0000x reference)
import functools
import numpy as np
import jax
import jax.numpy as jnp
from jax import lax
from jax.experimental import pallas as pl
from jax.experimental.pallas import tpu as pltpu
from jax.experimental.pallas import tpu_sc as plsc

_F = 200000
_E = 3 * _F
_NW = 16
_PER_W = _E // _NW
_W = 1024
_CAP = 37888
_M = 1 << 20
_TAGSH = 1 << 20
_R = 6

_i32 = jnp.int32
_f32 = jnp.float32


def _c(x):
    return int(np.int32(np.uint32(x)))


_KA = _c(0x9E3779B1)
_KB = _c(0x85EBCA6B)
_KC = _c(0x27D4EB2F)
_KD = _c(0x7FEB352D)
_KE = _c(0x846CA68B)

_mesh = plsc.VectorSubcoreMesh(
    core_axis_name="c", subcore_axis_name="s", num_cores=1)
_params = pltpu.CompilerParams(needs_layout_passes=False)


def _hash(c0, c1, r):
    x = c0 * _KA + c1 * _KB + r * _KC
    x = x ^ lax.shift_right_logical(x, _i32(16))
    x = x * _KD
    x = x ^ lax.shift_right_logical(x, _i32(13))
    x = x * _KE
    x = x ^ lax.shift_right_logical(x, _i32(16))
    return jnp.bitwise_and(x, _M - 1)


def _check(val, slot, saltv):
    return (val * _KD) ^ (slot * _KB) ^ saltv


_LANE = None


def _nwin_of(n):
    return lax.shift_right_logical(n + _i32(_W - 1), _i32(10))


def _body_a(round1, cs, c0s, c1s, gs, T,
            b_c0, b_c1, b_g, b_slot, b_val, b_cnt, s0, s1, s2):
    wid = lax.axis_index("s").astype(_i32)
    LANE = jnp.arange(16, dtype=_i32)
    if round1:
        n = _i32(_PER_W)
        r = _i32(1)
    else:
        pltpu.sync_copy(cs.at[wid], b_cnt)
        cv = b_cnt[...]
        n = cv[0]
        r = cv[2]
    tag = r * _TAGSH
    base = wid * _CAP

    def win(w, _):
        off = w * _W
        d1 = pltpu.async_copy(c0s.at[pl.ds(base + off, _W)], b_c0, s0)
        d2 = pltpu.async_copy(c1s.at[pl.ds(base + off, _W)], b_c1, s1)
        if not round1:
            pltpu.async_copy(gs.at[pl.ds(base + off, _W)], b_g, s2).wait()
        d1.wait()
        d2.wait()
        rem = n - off

        def vec(j, _):
            sl = pl.ds(j * 16, 16)
            lane = j * 16 + LANE
            mask = lane < rem
            sv = _hash(b_c0[sl], b_c1[sl], r)
            if round1:
                gidv = wid * _PER_W + off + lane
            else:
                gidv = b_g[sl]
            b_slot[sl] = jnp.where(mask, sv, -1)
            b_val[sl] = tag + gidv + 1
            return 0
        lax.fori_loop(_i32(0), _i32(_W // 16), vec, 0)
        pltpu.async_copy(
            b_val, T.at[plsc.Indices(b_slot, ignored_value=-1)], s0).wait()
        return 0
    lax.fori_loop(_i32(0), _nwin_of(n), win, 0)


def _body_bc(round1, c0f, c1f, saltr, cs, c0s, c1s, tvs, gs, T,
             T0a, T0b, T1a, T1b, nc0, nc1, ntv, ng, wsl, ncs,
             b_c0, b_c1, b_tv, b_g, b_slot, b_w, b_wk0, b_wk1,
             b_i1, b_i2, b_i3, b_i4, b_i5, b_val, b_chk, b_cnt, b_salt,
             s0, s1, s2, s3, s4, s5):
    wid = lax.axis_index("s").astype(_i32)
    LANE = jnp.arange(16, dtype=_i32)
    pltpu.sync_copy(saltr, b_salt)
    saltv = b_salt[...]
    if round1:
        n = _i32(_PER_W)
        r = _i32(1)
    else:
        pltpu.sync_copy(cs.at[wid], b_cnt)
        cv = b_cnt[...]
        n = cv[0]
        r = cv[2]
    tag = r * _TAGSH
    base = wid * _CAP

    def win(w, carry):
        newn, wn = carry
        off = w * _W
        d1 = pltpu.async_copy(c0s.at[pl.ds(base + off, _W)], b_c0, s0)
        d2 = pltpu.async_copy(c1s.at[pl.ds(base + off, _W)], b_c1, s1)
        d3 = pltpu.async_copy(tvs.at[pl.ds(base + off, _W)], b_tv, s2)
        if not round1:
            pltpu.async_copy(gs.at[pl.ds(base + off, _W)], b_g, s3).wait()
        d1.wait()
        d2.wait()
        d3.wait()
        rem = n - off

        def v0(j, _):
            sl = pl.ds(j * 16, 16)
            lane = j * 16 + LANE
            mask = lane < rem
            sv = _hash(b_c0[sl], b_c1[sl], r)
            b_slot[sl] = jnp.where(mask, sv, -1)
            if round1:
                b_g[sl] = wid * _PER_W + off + lane
            return 0
        lax.fori_loop(_i32(0), _i32(_W // 16), v0, 0)
        pltpu.async_copy(
            T.at[plsc.Indices(b_slot, ignored_value=-1)], b_w, s0).wait()

        def v1(j, _):
            sl = pl.ds(j * 16, 16)
            widv = b_w[sl] - (tag + 1)
            lane = j * 16 + LANE
            ok = (lane < rem) & (widv >= 0) & (widv < _E)
            b_i1[sl] = jnp.where(ok, widv, -1)
            return 0
        lax.fori_loop(_i32(0), _i32(_W // 16), v1, 0)
        d1 = pltpu.async_copy(
            c0f.at[plsc.Indices(b_i1, ignored_value=-1)], b_wk0, s1)
        d2 = pltpu.async_copy(
            c1f.at[plsc.Indices(b_i1, ignored_value=-1)], b_wk1, s2)
        d1.wait()
        d2.wait()

        def v2(j, carry2):
            nn, wc0 = carry2
            sl = pl.ds(j * 16, 16)
            c0v = b_c0[sl]
            c1v = b_c1[sl]
            tvv = b_tv[sl]
            gidv = b_g[sl]
            slotv = b_slot[sl]
            widv = b_i1[sl]
            lane = j * 16 + LANE
            mask = lane < rem
            res = (widv >= 0) & (b_wk0[sl] == c0v) & (b_wk1[sl] == c1v) & mask
            iswin = res & (widv == gidv)
            ordv = jnp.bitwise_and(tvv, 1)
            triv = lax.shift_right_logical(tvv, _i32(1))
            b_i2[sl] = jnp.where(res & (ordv == 0), slotv, -1)
            b_i3[sl] = jnp.where(res & (ordv == 1), slotv, -1)
            val = tag + triv + 1
            b_val[sl] = val
            b_chk[sl] = _check(val, slotv, saltv)
            keep = mask & jnp.logical_not(res)
            ki = jnp.where(keep, _i32(1), _i32(0))
            kci = plsc.cumsum(ki)
            b_i4[sl] = jnp.where(keep, wid * _CAP + nn + kci - 1, -1)
            nn = nn + jnp.sum(ki, dtype=_i32)
            wi = jnp.where(iswin, _i32(1), _i32(0))
            wci = plsc.cumsum(wi)
            b_i5[sl] = jnp.where(iswin, wid * _CAP + wc0 + wci - 1, -1)
            wc0 = wc0 + jnp.sum(wi, dtype=_i32)
            return (nn, wc0)
        newn, wn = lax.fori_loop(_i32(0), _i32(_W // 16), v2, (newn, wn))
        d1 = pltpu.async_copy(
            b_val, T0a.at[plsc.Indices(b_i2, ignored_value=-1)], s0)
        d2 = pltpu.async_copy(
            b_chk, T0b.at[plsc.Indices(b_i2, ignored_value=-1)], s1)
        d3 = pltpu.async_copy(
            b_val, T1a.at[plsc.Indices(b_i3, ignored_value=-1)], s2)
        d4 = pltpu.async_copy(
            b_chk, T1b.at[plsc.Indices(b_i3, ignored_value=-1)], s3)
        d1.wait()
        d2.wait()
        d1 = pltpu.async_copy(
            b_c0, nc0.at[plsc.Indices(b_i4, ignored_value=-1)], s0)
        d2 = pltpu.async_copy(
            b_c1, nc1.at[plsc.Indices(b_i4, ignored_value=-1)], s1)
        d3.wait()
        d4.wait()
        d3 = pltpu.async_copy(
            b_tv, ntv.at[plsc.Indices(b_i4, ignored_value=-1)], s2)
        d4 = pltpu.async_copy(
            b_g, ng.at[plsc.Indices(b_i4, ignored_value=-1)], s3)
        d5 = pltpu.async_copy(
            b_slot, wsl.at[plsc.Indices(b_i5, ignored_value=-1)], s4)
        d1.wait()
        d2.wait()
        d3.wait()
        d4.wait()
        d5.wait()
        return (newn, wn)
    newn, wn = lax.fori_loop(
        _i32(0), _nwin_of(n), win, (_i32(0), _i32(0)))
    b_cnt[...] = jnp.where(LANE == 0, newn,
                           jnp.where(LANE == 1, wn,
                                     jnp.where(LANE == 2, r + 1, 0)))
    pltpu.sync_copy(b_cnt, ncs.at[wid])


def _body_d(fnx, fny, fnz, saltr, cs, wsl,
            T0a, T0b, T1a, T1b, accs_in, accu_in, accs, accu,
            b_slot, b_a0, b_b0, b_a1, b_b1, b_i1, b_i2, b_i3,
            b_x0, b_y0, b_z0, b_x1, b_y1, b_z1, b_cnt, b_salt, b_acc,
            s0, s1, s2, s3, s4, s5):
    wid = lax.axis_index("s").astype(_i32)
    LANE = jnp.arange(16, dtype=_i32)
    pltpu.sync_copy(saltr, b_salt)
    saltv = b_salt[...]
    pltpu.sync_copy(cs.at[wid], b_cnt)
    cv = b_cnt[...]
    wn = cv[1]
    r = cv[2] - 1
    tag = r * _TAGSH
    pltpu.sync_copy(accs_in.at[wid], b_acc)
    pltpu.sync_copy(accu_in.at[wid], b_cnt)
    u0 = b_cnt[...][0]
    base = wid * _CAP

    def win(w, _):
        off = w * _W
        pltpu.sync_copy(wsl.at[pl.ds(base + off, _W)], b_slot)
        rem = wn - off

        def v1(j, _):
            sl = pl.ds(j * 16, 16)
            lane = j * 16 + LANE
            b_i1[sl] = jnp.where(lane < rem, b_slot[sl], -1)
            return 0
        lax.fori_loop(_i32(0), _i32(_W // 16), v1, 0)
        d1 = pltpu.async_copy(
            T0a.at[plsc.Indices(b_i1, ignored_value=-1)], b_a0, s0)
        d2 = pltpu.async_copy(
            T0b.at[plsc.Indices(b_i1, ignored_value=-1)], b_b0, s1)
        d3 = pltpu.async_copy(
            T1a.at[plsc.Indices(b_i1, ignored_value=-1)], b_a1, s2)
        d4 = pltpu.async_copy(
            T1b.at[plsc.Indices(b_i1, ignored_value=-1)], b_b1, s3)
        d1.wait()
        d2.wait()
        d3.wait()
        d4.wait()

        def v2(j, _):
            sl = pl.ds(j * 16, 16)
            slotv = b_i1[sl]
            a0 = b_a0[sl]
            a1 = b_a1[sl]
            t0 = a0 - (tag + 1)
            t1 = a1 - (tag + 1)
            ok0 = ((t0 >= 0) & (t0 < _F)
                   & (b_b0[sl] == _check(a0, slotv, saltv)))
            ok1 = ((t1 >= 0) & (t1 < _F)
                   & (b_b1[sl] == _check(a1, slotv, saltv)))
            lane = j * 16 + LANE
            mask = lane < rem
            b_i2[sl] = jnp.where(mask, jnp.where(ok0, t0, 0), -1)
            b_i3[sl] = jnp.where(mask, jnp.where(ok1, t1, 0), -1)
            return 0
        lax.fori_loop(_i32(0), _i32(_W // 16), v2, 0)
        d1 = pltpu.async_copy(
            fnx.at[plsc.Indices(b_i2, ignored_value=-1)], b_x0, s0)
        d2 = pltpu.async_copy(
            fny.at[plsc.Indices(b_i2, ignored_value=-1)], b_y0, s1)
        d3 = pltpu.async_copy(
            fnz.at[plsc.Indices(b_i2, ignored_value=-1)], b_z0, s2)
        d4 = pltpu.async_copy(
            fnx.at[plsc.Indices(b_i3, ignored_value=-1)], b_x1, s3)
        d5 = pltpu.async_copy(
            fny.at[plsc.Indices(b_i3, ignored_value=-1)], b_y1, s4)
        d6 = pltpu.async_copy(
            fnz.at[plsc.Indices(b_i3, ignored_value=-1)], b_z1, s5)
        d1.wait()
        d2.wait()
        d3.wait()
        d4.wait()
        d5.wait()
        d6.wait()

        def v3(j, _):
            sl = pl.ds(j * 16, 16)
            dot = (b_x0[sl] * b_x1[sl] + b_y0[sl] * b_y1[sl]
                   + b_z0[sl] * b_z1[sl])
            term = jnp.abs(1.0 - jnp.clip(dot, -1.0, 1.0))
            lane = j * 16 + LANE
            b_acc[...] = b_acc[...] + jnp.where(lane < rem, term, 0.0)
            return 0
        lax.fori_loop(_i32(0), _i32(_W // 16), v3, 0)
        return 0
    lax.fori_loop(_i32(0), _nwin_of(wn), win, 0)
    pltpu.sync_copy(b_acc, accs.at[wid])
    b_cnt[...] = jnp.where(LANE == 0, u0 + wn, 0)
    pltpu.sync_copy(b_cnt, accu.at[wid])


def _body_f(accs, accu, out, b_b, b_u, b_a, b_o, s0):
    wid = lax.axis_index("s").astype(_i32)
    LANE = jnp.arange(16, dtype=_i32)

    @pl.when(wid == 0)
    def _():
        b_o[...] = jnp.zeros((16,), _f32)
        b_u[...] = jnp.zeros((16,), _i32)

        def rb(i, _):
            pltpu.sync_copy(accs.at[i], b_a)
            pltpu.sync_copy(accu.at[i], b_b)
            b_o[...] = b_o[...] + b_a[...]
            b_u[...] = b_u[...] + b_b[...]
            return 0
        lax.fori_loop(_i32(0), _i32(_NW), rb, 0)
        stot = jnp.sum(b_o[...])
        utot = jnp.sum(jnp.where(LANE == 0, b_u[...], 0), dtype=_i32)
        b_o[...] = jnp.where(LANE == 0, stot, utot.astype(_f32))
        pltpu.sync_copy(b_o, out)


_vi = lambda: pltpu.VMEM((_W,), _i32)
_vf = lambda: pltpu.VMEM((_W,), _f32)
_v16i = lambda: pltpu.VMEM((16,), _i32)
_v16f = lambda: pltpu.VMEM((16,), _f32)
_sem = lambda: pltpu.SemaphoreType.DMA
_hbm_i = lambda n: pltpu.HBM((n,), _i32)


def _mk(body, out_types, n_scr_i, n_scr_f=0, n16i=0, n16f=0, nsem=3):
    scr = ([_vi() for _ in range(n_scr_i)] + [_vf() for _ in range(n_scr_f)]
           + [_v16i() for _ in range(n16i)] + [_v16f() for _ in range(n16f)]
           + [_sem() for _ in range(nsem)])
    return pl.kernel(body, out_type=out_types, mesh=_mesh,
                     compiler_params=_params, scratch_types=scr)


def _kernel_parts(face_normals, t_pos_idx):
    a = t_pos_idx.astype(_i32)
    v0, v1, v2 = a[:, 0], a[:, 1], a[:, 2]
    ea = jnp.stack([v0, v1, v2], axis=0).reshape(-1)
    eb = jnp.stack([v1, v2, v0], axis=0).reshape(-1)
    c0f = jnp.minimum(ea, eb)
    c1f = jnp.maximum(ea, eb)
    order = (ea > eb).astype(_i32)
    tri = jnp.tile(jnp.arange(_F, dtype=_i32), 3)
    tvf = 2 * tri + order
    salt = (jnp.sum(c0f, dtype=_i32) * _KA) ^ (jnp.sum(c1f, dtype=_i32) * _KB)
    saltr = jnp.zeros((16,), _i32) + salt

    def padw(x):
        return jnp.pad(x.reshape(_NW, _PER_W),
                       ((0, 0), (0, _CAP - _PER_W))).reshape(-1)

    fn = face_normals.astype(_f32)
    fnx, fny, fnz = fn[:, 0], fn[:, 1], fn[:, 2]
    c0s, c1s, tvs = padw(c0f), padw(c1f), padw(tvf)
    gs = c0s
    cnts = jnp.zeros((_NW, 16), _i32)
    accs = jnp.zeros((_NW, 16), _f32)
    accu = jnp.zeros((_NW, 16), _i32)
    NC = _NW * _CAP
    sds = jax.ShapeDtypeStruct

    for r in range(1, _R + 1):
        round1 = (r == 1)
        T = _mk(functools.partial(_body_a, round1),
                sds((_M,), _i32), 5, n16i=1)(cnts, c0s, c1s, gs)
        outs = _mk(functools.partial(_body_bc, round1),
                   tuple([sds((_M,), _i32)] * 4
                         + [sds((NC,), _i32)] * 5
                         + [sds((_NW, 16), _i32)]),
                   15, n16i=2, nsem=6)(
            c0f, c1f, saltr, cnts, c0s, c1s, tvs, gs, T)
        T0a, T0b, T1a, T1b, nc0, nc1, ntv, ng, wsl, ncs = outs
        accs, accu = _mk(_body_d,
                         (sds((_NW, 16), _f32), sds((_NW, 16), _i32)),
                         8, n_scr_f=6, n16i=2, n16f=1, nsem=6)(
            fnx, fny, fnz, saltr, ncs, wsl, T0a, T0b, T1a, T1b, accs, accu)
        c0s, c1s, tvs, gs, cnts = nc0, nc1, ntv, ng, ncs

    out = _mk(_body_f, sds((16,), _f32), 0, n16i=2, n16f=2, nsem=1)(
        accs, accu)
    return out


def kernel(face_normals, t_pos_idx):
    out = _kernel_parts(face_normals, t_pos_idx)
    return out[0] / out[1]


# speedup vs baseline: 1.0107x; 1.0107x over previous
import functools
import numpy as np
import jax
import jax.numpy as jnp
from jax import lax
from jax.experimental import pallas as pl
from jax.experimental.pallas import tpu as pltpu
from jax.experimental.pallas import tpu_sc as plsc

_F = 200000
_E = 3 * _F
_NW = 16
_PER_W = _E // _NW
_W = 1024
_CAP = 37888
_M = 1 << 20
_TAGSH = 1 << 20
_R = 6

_i32 = jnp.int32
_f32 = jnp.float32


def _c(x):
    return int(np.int32(np.uint32(x)))


_KA = _c(0x9E3779B1)
_KB = _c(0x85EBCA6B)
_KC = _c(0x27D4EB2F)
_KD = _c(0x7FEB352D)
_KE = _c(0x846CA68B)

_mesh = plsc.VectorSubcoreMesh(
    core_axis_name="c", subcore_axis_name="s", num_cores=1)
_params = pltpu.CompilerParams(needs_layout_passes=False)


def _hash(c0, c1, r):
    x = c0 * _KA + c1 * _KB + r * _KC
    x = x ^ lax.shift_right_logical(x, _i32(16))
    x = x * _KD
    x = x ^ lax.shift_right_logical(x, _i32(13))
    x = x * _KE
    x = x ^ lax.shift_right_logical(x, _i32(16))
    return jnp.bitwise_and(x, _M - 1)


def _check(val, slot, saltv):
    return (val * _KD) ^ (slot * _KB) ^ saltv


def _nwin_of(n):
    return lax.shift_right_logical(n + _i32(_W - 1), _i32(10))


def _body_a(round1, cs, c0s, c1s, gs, T,
            b_c0, b_c1, b_g, b_slot, b_val, b_cnt, s0, s1, s2):
    wid = lax.axis_index("s").astype(_i32)
    LANE = jnp.arange(16, dtype=_i32)
    if round1:
        n = _i32(_PER_W)
        r = _i32(1)
    else:
        pltpu.sync_copy(cs.at[wid], b_cnt)
        cv = b_cnt[...]
        n = cv[0]
        r = cv[2]
    tag = r * _TAGSH
    base = wid * _CAP

    def win(w, _):
        off = w * _W
        d1 = pltpu.async_copy(c0s.at[pl.ds(base + off, _W)], b_c0, s0)
        d2 = pltpu.async_copy(c1s.at[pl.ds(base + off, _W)], b_c1, s1)
        if not round1:
            pltpu.async_copy(gs.at[pl.ds(base + off, _W)], b_g, s2).wait()
        d1.wait()
        d2.wait()
        rem = n - off

        def vec(j, _):
            sl = pl.ds(j * 16, 16)
            lane = j * 16 + LANE
            mask = lane < rem
            sv = _hash(b_c0[sl], b_c1[sl], r)
            if round1:
                gidv = wid * _PER_W + off + lane
            else:
                gidv = b_g[sl]
            b_slot[sl] = jnp.where(mask, sv, -1)
            b_val[sl] = tag + gidv + 1
            return 0
        lax.fori_loop(_i32(0), _i32(_W // 16), vec, 0)
        pltpu.async_copy(
            b_val, T.at[plsc.Indices(b_slot, ignored_value=-1)], s0).wait()
        return 0
    lax.fori_loop(_i32(0), _nwin_of(n), win, 0)


def _body_bc(round1, c0f, c1f, saltr, cs, c0s, c1s, tvs, gs, T,
             T0a, T0b, T1a, T1b, nc0, nc1, ntv, ng, wsl, ncs,
             b_c0, b_c1, b_tv, b_g, b_slot, b_w, b_wk0, b_wk1,
             b_i1, b_i2, b_i3, b_i4, b_i5, b_val, b_chk, b_cnt, b_salt,
             s0, s1, s2, s3, s4, s5):
    wid = lax.axis_index("s").astype(_i32)
    LANE = jnp.arange(16, dtype=_i32)
    pltpu.sync_copy(saltr, b_salt)
    saltv = b_salt[...]
    if round1:
        n = _i32(_PER_W)
        r = _i32(1)
    else:
        pltpu.sync_copy(cs.at[wid], b_cnt)
        cv = b_cnt[...]
        n = cv[0]
        r = cv[2]
    tag = r * _TAGSH
    base = wid * _CAP

    def win(w, carry):
        newn, wn = carry
        off = w * _W
        d1 = pltpu.async_copy(c0s.at[pl.ds(base + off, _W)], b_c0, s0)
        d2 = pltpu.async_copy(c1s.at[pl.ds(base + off, _W)], b_c1, s1)
        d3 = pltpu.async_copy(tvs.at[pl.ds(base + off, _W)], b_tv, s2)
        if not round1:
            pltpu.async_copy(gs.at[pl.ds(base + off, _W)], b_g, s3).wait()
        d1.wait()
        d2.wait()
        d3.wait()
        rem = n - off

        def v0(j, _):
            sl = pl.ds(j * 16, 16)
            lane = j * 16 + LANE
            mask = lane < rem
            sv = _hash(b_c0[sl], b_c1[sl], r)
            b_slot[sl] = jnp.where(mask, sv, -1)
            if round1:
                b_g[sl] = wid * _PER_W + off + lane
            return 0
        lax.fori_loop(_i32(0), _i32(_W // 16), v0, 0)
        pltpu.async_copy(
            T.at[plsc.Indices(b_slot, ignored_value=-1)], b_w, s0).wait()

        def v1(j, _):
            sl = pl.ds(j * 16, 16)
            widv = b_w[sl] - (tag + 1)
            lane = j * 16 + LANE
            ok = (lane < rem) & (widv >= 0) & (widv < _E)
            b_i1[sl] = jnp.where(ok, widv, -1)
            return 0
        lax.fori_loop(_i32(0), _i32(_W // 16), v1, 0)
        d1 = pltpu.async_copy(
            c0f.at[plsc.Indices(b_i1, ignored_value=-1)], b_wk0, s1)
        d2 = pltpu.async_copy(
            c1f.at[plsc.Indices(b_i1, ignored_value=-1)], b_wk1, s2)
        d1.wait()
        d2.wait()

        def v2(j, carry2):
            nn, wc0 = carry2
            sl = pl.ds(j * 16, 16)
            c0v = b_c0[sl]
            c1v = b_c1[sl]
            tvv = b_tv[sl]
            gidv = b_g[sl]
            slotv = b_slot[sl]
            widv = b_i1[sl]
            lane = j * 16 + LANE
            mask = lane < rem
            res = (widv >= 0) & (b_wk0[sl] == c0v) & (b_wk1[sl] == c1v) & mask
            iswin = res & (widv == gidv)
            ordv = jnp.bitwise_and(tvv, 1)
            triv = lax.shift_right_logical(tvv, _i32(1))
            b_i2[sl] = jnp.where(res & (ordv == 0), slotv, -1)
            b_i3[sl] = jnp.where(res & (ordv == 1), slotv, -1)
            val = tag + triv + 1
            b_val[sl] = val
            b_chk[sl] = _check(val, slotv, saltv)
            keep = mask & jnp.logical_not(res)
            ki = jnp.where(keep, _i32(1), _i32(0))
            kci = plsc.cumsum(ki)
            b_i4[sl] = jnp.where(keep, wid * _CAP + nn + kci - 1, -1)
            nn = nn + jnp.sum(ki, dtype=_i32)
            wi = jnp.where(iswin, _i32(1), _i32(0))
            wci = plsc.cumsum(wi)
            b_i5[sl] = jnp.where(iswin, wid * _CAP + wc0 + wci - 1, -1)
            wc0 = wc0 + jnp.sum(wi, dtype=_i32)
            return (nn, wc0)
        newn, wn = lax.fori_loop(_i32(0), _i32(_W // 16), v2, (newn, wn))
        d1 = pltpu.async_copy(
            b_val, T0a.at[plsc.Indices(b_i2, ignored_value=-1)], s0)
        d2 = pltpu.async_copy(
            b_chk, T0b.at[plsc.Indices(b_i2, ignored_value=-1)], s1)
        d3 = pltpu.async_copy(
            b_val, T1a.at[plsc.Indices(b_i3, ignored_value=-1)], s2)
        d4 = pltpu.async_copy(
            b_chk, T1b.at[plsc.Indices(b_i3, ignored_value=-1)], s3)
        d1.wait()
        d2.wait()
        d1 = pltpu.async_copy(
            b_c0, nc0.at[plsc.Indices(b_i4, ignored_value=-1)], s0)
        d2 = pltpu.async_copy(
            b_c1, nc1.at[plsc.Indices(b_i4, ignored_value=-1)], s1)
        d3.wait()
        d4.wait()
        d3 = pltpu.async_copy(
            b_tv, ntv.at[plsc.Indices(b_i4, ignored_value=-1)], s2)
        d4 = pltpu.async_copy(
            b_g, ng.at[plsc.Indices(b_i4, ignored_value=-1)], s3)
        d5 = pltpu.async_copy(
            b_slot, wsl.at[plsc.Indices(b_i5, ignored_value=-1)], s4)
        d1.wait()
        d2.wait()
        d3.wait()
        d4.wait()
        d5.wait()
        return (newn, wn)
    newn, wn = lax.fori_loop(
        _i32(0), _nwin_of(n), win, (_i32(0), _i32(0)))
    b_cnt[...] = jnp.where(LANE == 0, newn,
                           jnp.where(LANE == 1, wn,
                                     jnp.where(LANE == 2, r + 1, 0)))
    pltpu.sync_copy(b_cnt, ncs.at[wid])


def _body_d(fnx, fny, fnz, saltr, cs, wsl,
            T0a, T0b, T1a, T1b, accs_in, accu_in, accs, accu,
            b_slot, b_a0, b_b0, b_a1, b_b1, b_i1, b_i2, b_i3,
            b_x0, b_y0, b_z0, b_x1, b_y1, b_z1, b_cnt, b_salt, b_acc,
            s0, s1, s2, s3, s4, s5):
    wid = lax.axis_index("s").astype(_i32)
    LANE = jnp.arange(16, dtype=_i32)
    pltpu.sync_copy(saltr, b_salt)
    saltv = b_salt[...]
    pltpu.sync_copy(cs.at[wid], b_cnt)
    cv = b_cnt[...]
    wn = cv[1]
    r = cv[2] - 1
    tag = r * _TAGSH
    pltpu.sync_copy(accs_in.at[wid], b_acc)
    pltpu.sync_copy(accu_in.at[wid], b_cnt)
    u0 = b_cnt[...][0]
    base = wid * _CAP

    def win(w, _):
        off = w * _W
        pltpu.sync_copy(wsl.at[pl.ds(base + off, _W)], b_slot)
        rem = wn - off

        def v1(j, _):
            sl = pl.ds(j * 16, 16)
            lane = j * 16 + LANE
            b_i1[sl] = jnp.where(lane < rem, b_slot[sl], -1)
            return 0
        lax.fori_loop(_i32(0), _i32(_W // 16), v1, 0)
        d1 = pltpu.async_copy(
            T0a.at[plsc.Indices(b_i1, ignored_value=-1)], b_a0, s0)
        d2 = pltpu.async_copy(
            T0b.at[plsc.Indices(b_i1, ignored_value=-1)], b_b0, s1)
        d3 = pltpu.async_copy(
            T1a.at[plsc.Indices(b_i1, ignored_value=-1)], b_a1, s2)
        d4 = pltpu.async_copy(
            T1b.at[plsc.Indices(b_i1, ignored_value=-1)], b_b1, s3)
        d1.wait()
        d2.wait()
        d3.wait()
        d4.wait()

        def v2(j, _):
            sl = pl.ds(j * 16, 16)
            slotv = b_i1[sl]
            a0 = b_a0[sl]
            a1 = b_a1[sl]
            t0 = a0 - (tag + 1)
            t1 = a1 - (tag + 1)
            ok0 = ((t0 >= 0) & (t0 < _F)
                   & (b_b0[sl] == _check(a0, slotv, saltv)))
            ok1 = ((t1 >= 0) & (t1 < _F)
                   & (b_b1[sl] == _check(a1, slotv, saltv)))
            lane = j * 16 + LANE
            mask = lane < rem
            b_i2[sl] = jnp.where(mask, jnp.where(ok0, t0, 0), -1)
            b_i3[sl] = jnp.where(mask, jnp.where(ok1, t1, 0), -1)
            return 0
        lax.fori_loop(_i32(0), _i32(_W // 16), v2, 0)
        d1 = pltpu.async_copy(
            fnx.at[plsc.Indices(b_i2, ignored_value=-1)], b_x0, s0)
        d2 = pltpu.async_copy(
            fny.at[plsc.Indices(b_i2, ignored_value=-1)], b_y0, s1)
        d3 = pltpu.async_copy(
            fnz.at[plsc.Indices(b_i2, ignored_value=-1)], b_z0, s2)
        d4 = pltpu.async_copy(
            fnx.at[plsc.Indices(b_i3, ignored_value=-1)], b_x1, s3)
        d5 = pltpu.async_copy(
            fny.at[plsc.Indices(b_i3, ignored_value=-1)], b_y1, s4)
        d6 = pltpu.async_copy(
            fnz.at[plsc.Indices(b_i3, ignored_value=-1)], b_z1, s5)
        d1.wait()
        d2.wait()
        d3.wait()
        d4.wait()
        d5.wait()
        d6.wait()

        def v3(j, _):
            sl = pl.ds(j * 16, 16)
            dot = (b_x0[sl] * b_x1[sl] + b_y0[sl] * b_y1[sl]
                   + b_z0[sl] * b_z1[sl])
            term = jnp.abs(1.0 - jnp.clip(dot, -1.0, 1.0))
            lane = j * 16 + LANE
            b_acc[...] = b_acc[...] + jnp.where(lane < rem, term, 0.0)
            return 0
        lax.fori_loop(_i32(0), _i32(_W // 16), v3, 0)
        return 0
    lax.fori_loop(_i32(0), _nwin_of(wn), win, 0)
    pltpu.sync_copy(b_acc, accs.at[wid])
    b_cnt[...] = jnp.where(LANE == 0, u0 + wn, 0)
    pltpu.sync_copy(b_cnt, accu.at[wid])


def _body_f(accs, accu, out, b_b, b_u, b_a, b_o, s0):
    wid = lax.axis_index("s").astype(_i32)
    LANE = jnp.arange(16, dtype=_i32)

    @pl.when(wid == 0)
    def _():
        b_o[...] = jnp.zeros((16,), _f32)
        b_u[...] = jnp.zeros((16,), _i32)

        def rb(i, _):
            pltpu.sync_copy(accs.at[i], b_a)
            pltpu.sync_copy(accu.at[i], b_b)
            b_o[...] = b_o[...] + b_a[...]
            b_u[...] = b_u[...] + b_b[...]
            return 0
        lax.fori_loop(_i32(0), _i32(_NW), rb, 0)
        stot = jnp.sum(b_o[...])
        utot = jnp.sum(jnp.where(LANE == 0, b_u[...], 0), dtype=_i32)
        b_o[...] = jnp.where(LANE == 0, stot, utot.astype(_f32))
        pltpu.sync_copy(b_o, out)


_vi = lambda: pltpu.VMEM((_W,), _i32)
_vf = lambda: pltpu.VMEM((_W,), _f32)
_v16i = lambda: pltpu.VMEM((16,), _i32)
_v16f = lambda: pltpu.VMEM((16,), _f32)
_sem = lambda: pltpu.SemaphoreType.DMA
_hbm_i = lambda n: pltpu.HBM((n,), _i32)


def _mk(body, out_types, n_scr_i, n_scr_f=0, n16i=0, n16f=0, nsem=3):
    scr = ([_vi() for _ in range(n_scr_i)] + [_vf() for _ in range(n_scr_f)]
           + [_v16i() for _ in range(n16i)] + [_v16f() for _ in range(n16f)]
           + [_sem() for _ in range(nsem)])
    return pl.kernel(body, out_type=out_types, mesh=_mesh,
                     compiler_params=_params, scratch_types=scr)


def _kernel_parts(face_normals, t_pos_idx):
    a = t_pos_idx.astype(_i32)
    v0, v1, v2 = a[:, 0], a[:, 1], a[:, 2]
    ea = jnp.stack([v0, v1, v2], axis=0).reshape(-1)
    eb = jnp.stack([v1, v2, v0], axis=0).reshape(-1)
    c0f = jnp.minimum(ea, eb)
    c1f = jnp.maximum(ea, eb)
    order = (ea > eb).astype(_i32)
    tri = jnp.tile(jnp.arange(_F, dtype=_i32), 3)
    tvf = 2 * tri + order
    salt = (jnp.sum(c0f, dtype=_i32) * _KA) ^ (jnp.sum(c1f, dtype=_i32) * _KB)
    saltr = jnp.zeros((16,), _i32) + salt

    def padw(x):
        return jnp.pad(x.reshape(_NW, _PER_W),
                       ((0, 0), (0, _CAP - _PER_W))).reshape(-1)

    fn = face_normals.astype(_f32)
    fnx, fny, fnz = fn[:, 0], fn[:, 1], fn[:, 2]
    c0s, c1s, tvs = padw(c0f), padw(c1f), padw(tvf)
    gs = c0s
    cnts = jnp.zeros((_NW, 16), _i32)
    accs = jnp.zeros((_NW, 16), _f32)
    accu = jnp.zeros((_NW, 16), _i32)
    NC = _NW * _CAP
    sds = jax.ShapeDtypeStruct

    for r in range(1, _R + 1):
        round1 = (r == 1)
        T = _mk(functools.partial(_body_a, round1),
                sds((_M,), _i32), 5, n16i=1)(cnts, c0s, c1s, gs)
        outs = _mk(functools.partial(_body_bc, round1),
                   tuple([sds((_M,), _i32)] * 4
                         + [sds((NC,), _i32)] * 5
                         + [sds((_NW, 16), _i32)]),
                   15, n16i=2, nsem=6)(
            c0f, c1f, saltr, cnts, c0s, c1s, tvs, gs, T)
        T0a, T0b, T1a, T1b, nc0, nc1, ntv, ng, wsl, ncs = outs
        accs, accu = _mk(_body_d,
                         (sds((_NW, 16), _f32), sds((_NW, 16), _i32)),
                         8, n_scr_f=6, n16i=2, n16f=1, nsem=6)(
            fnx, fny, fnz, saltr, ncs, wsl, T0a, T0b, T1a, T1b, accs, accu)
        c0s, c1s, tvs, gs, cnts = nc0, nc1, ntv, ng, ncs

    out = _mk(_body_f, sds((16,), _f32), 0, n16i=2, n16f=2, nsem=1)(
        accs, accu)
    return out


def kernel(face_normals, t_pos_idx):
    out = _kernel_parts(face_normals, t_pos_idx)
    return out[0] / out[1]
